# baseline (speedup 1.0000x reference)
import functools
import math

import jax
import jax.numpy as jnp
from jax import lax
from jax.experimental import pallas as pl
from jax.experimental.pallas import tpu as pltpu
from jax.experimental.pallas import tpu_sc as plsc

N = 10000
E = 160000
HD = 128
MSG = 24
EE = 16
THETA = 10000.0

NC = 2
NS = 16
NW = NC * NS
CH = 128
NCHUNK = 40
EP = NW * NCHUNK * CH
EPW = EP // NW
NP = 10240
F32 = jnp.float32


def _silu(x):
    return x / (1.0 + jnp.exp(-x))


def _dot(a, b):
    return jnp.dot(a.astype(jnp.bfloat16), b.astype(jnp.bfloat16),
                   preferred_element_type=F32)


def _bf(x):
    return x.astype(jnp.bfloat16).astype(F32)


_PIO2_HI = 1.5707963705062866
_PIO2_LO = -4.371138828673793e-08


def _fast_cos(x):
    q = jnp.floor(x * (2.0 / math.pi) + 0.5)
    r = (x - q * _PIO2_HI) - q * _PIO2_LO
    r2 = r * r
    c = 1.0 + r2 * (-0.5 + r2 * (4.16666679e-2 + r2 * (-1.38888881e-3
        + r2 * 2.48015873e-5)))
    s = r * (1.0 + r2 * (-1.66666672e-1 + r2 * (8.33333358e-3
        + r2 * (-1.98412701e-4 + r2 * 2.75573188e-6))))
    qi = q.astype(jnp.int32)
    val = jnp.where((qi & 1) == 1, s, c)
    return jnp.where(((qi + 1) & 2) == 0, val, -val)


def _prep_body(h_ref, p32_ref, wemb_ref, bemb_ref, w1a_ref, w1b_ref,
               h1_ref, a_ref, b_ref):
    h1 = _dot(h_ref[...], wemb_ref[...]) + bemb_ref[...]
    h1_ref[...] = h1
    p32 = p32_ref[...]
    a_ref[...] = p32 - _dot(h1, w1a_ref[...])
    b_ref[...] = p32 + _dot(h1, w1b_ref[...])


def _edge0_body(base, g_ref, w1e_ref, w1r_ref, b1_ref, w2_ref, b2_ref,
                c1_ref, c1b_ref, c2_ref, out_ref, ea_ref):
    g = g_ref[...]
    lane32 = lax.broadcasted_iota(jnp.int32, (1, 32), 1)
    posm = (lane32 >= 28) & (lane32 < 31)
    gp = jnp.where(posm, g, 0.0)
    radial = jnp.sum(gp * gp, axis=1, keepdims=True)
    dist = jnp.sqrt(radial)
    lane16 = lax.broadcasted_iota(jnp.int32, (1, 16), 1)
    f16 = jnp.exp(-math.log(THETA) * ((lane16 // 2) * 2).astype(F32) / EE)
    ang = dist * f16
    off = jnp.where(lane16 % 2 == 0, jnp.float32(math.pi / 2), 0.0)
    ea = _fast_cos(ang - off)
    ea_ref[...] = ea.astype(jnp.bfloat16)
    pre = (g + _dot(ea, w1e_ref[...]) + _bf(radial) * _bf(w1r_ref[...])
           + b1_ref[...])
    msg = _silu(_dot(_silu(pre), w2_ref[...]) + b2_ref[...])
    t = _silu(_dot(msg, c1_ref[...]) + c1b_ref[...])
    w = jnp.sum(_bf(t) * _bf(c2_ref[...]), axis=1, keepdims=True)
    out = (jnp.where(lane32 < 24, msg, 0.0)
           + jnp.where(lane32 == 24, 1.0, 0.0)
           + jnp.where(posm, w * g, 0.0))
    gid = (lax.broadcasted_iota(jnp.int32, (g.shape[0], 1), 0)
           + pl.program_id(0) * g.shape[0] + base)
    out_ref[...] = jnp.where(gid < E, out, 0.0)


def _edge1_body(base, g_ref, ea_ref, w1e_ref, w1r_ref, b1_ref, w2_ref,
                b2_ref, out_ref):
    g = g_ref[...]
    lane32 = lax.broadcasted_iota(jnp.int32, (1, 32), 1)
    posm = (lane32 >= 28) & (lane32 < 31)
    gp = jnp.where(posm, g, 0.0)
    radial = jnp.sum(gp * gp, axis=1, keepdims=True)
    pre = (g + _dot(ea_ref[...], w1e_ref[...])
           + _bf(radial) * _bf(w1r_ref[...]) + b1_ref[...])
    msg = _silu(_dot(_silu(pre), w2_ref[...]) + b2_ref[...])
    out = (jnp.where(lane32 < 24, msg, 0.0)
           + jnp.where(lane32 == 24, 1.0, 0.0))
    gid = (lax.broadcasted_iota(jnp.int32, (g.shape[0], 1), 0)
           + pl.program_id(0) * g.shape[0] + base)
    out_ref[...] = jnp.where(gid < E, out, 0.0)


def _node_core(hh, s, n1h, n1m, b1n, n2, b2n, gln, bln):
    lane32 = lax.broadcasted_iota(jnp.int32, (1, 32), 1)
    cnt = jnp.sum(jnp.where(lane32 == 24, s, 0.0), axis=1, keepdims=True)
    denom = jnp.maximum(cnt, 1.0)
    magg = jnp.where(lane32 < 24, s, 0.0) / denom
    u = _dot(hh, n1h) + _dot(magg, n1m) + b1n
    hn = hh + _dot(_silu(u), n2) + b2n
    m = jnp.mean(hn, axis=1, keepdims=True)
    v = jnp.mean((hn - m) ** 2, axis=1, keepdims=True)
    return (hn - m) / jnp.sqrt(v + 1e-5) * gln + bln


def _node0_body(h_ref, s0_ref, s1_ref, s2_ref, s3_ref, p32_ref, n1h_ref,
                n1m_ref, b1n_ref, n2_ref, b2n_ref, gln_ref, bln_ref,
                w1a_ref, w1b_ref, hout_ref, p32n_ref, a_ref, b_ref):
    s = (s0_ref[...] + s1_ref[...]) + (s2_ref[...] + s3_ref[...])
    ho = _node_core(h_ref[...], s, n1h_ref[...], n1m_ref[...], b1n_ref[...],
                    n2_ref[...], b2n_ref[...], gln_ref[...], bln_ref[...])
    hout_ref[...] = ho
    lane32 = lax.broadcasted_iota(jnp.int32, (1, 32), 1)
    cadd = jnp.where((lane32 >= 28) & (lane32 < 31), s, 0.0)
    p32n = p32_ref[...] + cadd
    p32n_ref[...] = p32n
    a_ref[...] = p32n - _dot(ho, w1a_ref[...])
    b_ref[...] = p32n + _dot(ho, w1b_ref[...])


def _node1_body(h_ref, s0_ref, s1_ref, s2_ref, s3_ref, n1h_ref, n1m_ref,
                b1n_ref, n2_ref, b2n_ref, gln_ref, bln_ref, hout_ref):
    s = (s0_ref[...] + s1_ref[...]) + (s2_ref[...] + s3_ref[...])
    hout_ref[...] = _node_core(h_ref[...], s, n1h_ref[...], n1m_ref[...],
                               b1n_ref[...], n2_ref[...], b2n_ref[...],
                               gln_ref[...], bln_ref[...])


def _full(shape):
    return pl.BlockSpec(shape, lambda i: (0,) * len(shape))


def _rows(bs, ncols):
    return pl.BlockSpec((bs, ncols), lambda i: (i, 0))


@functools.cache
def _sc_mesh():
    return plsc.VectorSubcoreMesh(core_axis_name="c", subcore_axis_name="s",
                                  num_cores=NC, num_subcores=NS)


@functools.cache
def _sc_gather_kernel(nchunk, cbase):
    epw = nchunk * CH

    def body(atab, btab, rowr, colr, g_out, idxr, idxc, bufa, bufb,
             sema, semb):
        cid = lax.axis_index("c")
        sid = lax.axis_index("s")
        wid = sid * NC + cid
        pltpu.sync_copy(rowr.at[pl.ds(cbase + wid * nchunk, nchunk)], idxr)
        pltpu.sync_copy(colr.at[pl.ds(cbase + wid * nchunk, nchunk)], idxc)

        def fire(j, b):
            pltpu.async_copy(atab.at[idxc.at[j]], bufa[b], sema[b])
            pltpu.async_copy(btab.at[idxr.at[j]], bufb[b], semb[b])

        def process(j, b):
            pltpu.make_async_copy(atab.at[idxc.at[j]], bufa[b],
                                  sema[b]).wait()
            pltpu.make_async_copy(btab.at[idxr.at[j]], bufb[b],
                                  semb[b]).wait()

            def sub(i, c2):
                bufb[b][i, 0:16] = bufb[b][i, 0:16] - bufa[b][i, 0:16]
                bufb[b][i, 16:32] = bufb[b][i, 16:32] - bufa[b][i, 16:32]
                return c2

            lax.fori_loop(0, CH, sub, 0, unroll=8)
            pltpu.sync_copy(bufb[b], g_out.at[pl.ds(wid * epw + j * CH, CH)])

        for b in range(4):
            fire(b, b)

        def group(t, carry):
            for b in range(4):
                j = t * 4 + b
                process(j, b)

                @pl.when(j + 4 < nchunk)
                def _():
                    fire(j + 4, b)

            return carry

        lax.fori_loop(0, nchunk // 4, group, 0)

    return pl.kernel(
        body,
        out_type=jax.ShapeDtypeStruct((NW * epw, 32), F32),
        mesh=_sc_mesh(),
        compiler_params=pltpu.CompilerParams(use_tc_tiling_on_sc=False),
        scratch_types=[
            pltpu.VMEM((nchunk, CH), jnp.int32),
            pltpu.VMEM((nchunk, CH), jnp.int32),
            [pltpu.VMEM((CH, 32), F32)] * 4,
            [pltpu.VMEM((CH, 32), F32)] * 4,
            [pltpu.SemaphoreType.DMA] * 4,
            [pltpu.SemaphoreType.DMA] * 4,
        ],
    )


@functools.cache
def _sc_scatter_kernel(nchunk, cbase):
    epw = nchunk * CH

    def body(m_hbm, colr, s_out, idxc, mbuf, zbuf, acc, msem):
        cid = lax.axis_index("c")
        sid = lax.axis_index("s")
        wid = sid * NC + cid
        rpt = NP // NS

        def zb(i, carry):
            zbuf[i, 0:16] = jnp.zeros((16,), F32)
            zbuf[i, 16:32] = jnp.zeros((16,), F32)
            return carry

        lax.fori_loop(0, rpt, zb, 0, unroll=8)
        pltpu.sync_copy(zbuf, acc.at[pl.ds(sid * rpt, rpt)])
        pltpu.sync_copy(colr.at[pl.ds(cbase + wid * nchunk, nchunk)], idxc)
        plsc.subcore_barrier()

        def fire(j, b):
            pltpu.async_copy(m_hbm.at[pl.ds(wid * epw + j * CH, CH)],
                             mbuf[b], msem[b])

        def process(j, b):
            pltpu.make_async_copy(
                m_hbm.at[pl.ds(wid * epw + j * CH, CH)], mbuf[b],
                msem[b]).wait()
            pltpu.sync_copy(mbuf[b], acc.at[idxc.at[j]], add=True)

        fire(0, 0)
        fire(1, 1)

        def pair(t, carry):
            j0 = t * 2
            process(j0, 0)

            @pl.when(j0 + 2 < nchunk)
            def _():
                fire(j0 + 2, 0)

            process(j0 + 1, 1)

            @pl.when(j0 + 3 < nchunk)
            def _():
                fire(j0 + 3, 1)

            return carry

        lax.fori_loop(0, nchunk // 2, pair, 0)
        plsc.subcore_barrier()
        pltpu.sync_copy(acc.at[pl.ds(sid * rpt, rpt)], zbuf)
        pltpu.sync_copy(zbuf, s_out.at[cid, pl.ds(sid * rpt, rpt)])

    return pl.kernel(
        body,
        out_type=jax.ShapeDtypeStruct((NC, NP, 32), F32),
        mesh=_sc_mesh(),
        compiler_params=pltpu.CompilerParams(use_tc_tiling_on_sc=False),
        scratch_types=[
            pltpu.VMEM((nchunk, CH), jnp.int32),
            [pltpu.VMEM((CH, 32), F32)] * 2,
            pltpu.VMEM((NP // NS, 32), F32),
            pltpu.VMEM_SHARED((NP, 32), F32),
            [pltpu.SemaphoreType.DMA] * 2,
        ],
    )


def _padc(w, cols=32):
    return jnp.zeros((w.shape[0], cols), F32).at[:, : w.shape[1]].set(w)


def _padrc(w, rows=32, cols=32):
    return (jnp.zeros((rows, cols), F32)
            .at[: w.shape[0], : w.shape[1]].set(w))


def _padv(b, cols=32):
    return jnp.zeros((1, cols), F32).at[0, : b.shape[0]].set(b)


def kernel(h, pos, edge_index, params):
    row = edge_index[0]
    col = edge_index[1]
    rowr = jnp.zeros((EP,), jnp.int32).at[:E].set(row).reshape(
        NW * NCHUNK, CH)
    colr = jnp.zeros((EP,), jnp.int32).at[:E].set(col).reshape(
        NW * NCHUNK, CH)
    p32 = jnp.zeros((N, 32), F32).at[:, 28:31].set(pos)

    lp = params["layers"]
    emb_w = params["node_emb"]["W"]
    emb_b = params["node_emb"]["b"].reshape(1, HD)

    def layer_weights(l):
        lay = lp[l]
        w1 = lay["msg1"]["W"]
        d = {
            "w1a": _padc(w1[:HD]),
            "w1b": _padc(w1[HD:2 * HD]),
            "w1e": _padc(w1[2 * HD:2 * HD + EE]),
            "w1r": _padv(w1[2 * HD + EE]),
            "b1": _padv(lay["msg1"]["b"]),
            "w2": _padrc(lay["msg2"]["W"]),
            "b2": _padv(lay["msg2"]["b"]),
            "n1h": _padc(lay["node1"]["W"][:HD]),
            "n1m": _padrc(lay["node1"]["W"][HD:]),
            "b1n": _padv(lay["node1"]["b"]),
            "n2": jnp.zeros((32, HD), F32).at[:MSG].set(lay["node2"]["W"]),
            "b2n": lay["node2"]["b"].reshape(1, HD),
            "gln": lay["ln_g"].reshape(1, HD),
            "bln": lay["ln_b"].reshape(1, HD),
        }
        if "coord1" in lay:
            d["c1"] = _padrc(lay["coord1"]["W"])
            d["c1b"] = _padv(lay["coord1"]["b"])
            d["c2"] = _padv(lay["coord2"]["W"][:, 0])
        return d

    w0 = layer_weights(0)
    w1l = layer_weights(1)

    bn = 1000
    gn = N // bn
    be = 2048
    EH = EP // 2
    geh = EH // be
    NCH = NCHUNK // 2
    CB = NW * NCH

    h1, a0, b0 = pl.pallas_call(
        _prep_body,
        grid=(gn,),
        in_specs=[_rows(bn, HD), _rows(bn, 32), _full((HD, HD)),
                  _full((1, HD)), _full((HD, 32)), _full((HD, 32))],
        out_specs=[_rows(bn, HD), _rows(bn, 32), _rows(bn, 32)],
        out_shape=[jax.ShapeDtypeStruct((N, HD), F32),
                   jax.ShapeDtypeStruct((N, 32), F32),
                   jax.ShapeDtypeStruct((N, 32), F32)],
    )(h, p32, emb_w, emb_b, w0["w1a"], w0["w1b"])

    def edge0_half(g_half, base):
        return pl.pallas_call(
            functools.partial(_edge0_body, base),
            grid=(geh,),
            in_specs=[_rows(be, 32), _full((EE, 32)), _full((1, 32)),
                      _full((1, 32)), _full((32, 32)), _full((1, 32)),
                      _full((32, 32)), _full((1, 32)), _full((1, 32))],
            out_specs=[_rows(be, 32), _rows(be, EE)],
            out_shape=[jax.ShapeDtypeStruct((EH, 32), F32),
                       jax.ShapeDtypeStruct((EH, EE), jnp.bfloat16)],
        )(g_half, w0["w1e"], w0["w1r"], w0["b1"], w0["w2"], w0["b2"],
          w0["c1"], w0["c1b"], w0["c2"])

    def edge1_half(g_half, ea_half, base):
        return pl.pallas_call(
            functools.partial(_edge1_body, base),
            grid=(geh,),
            in_specs=[_rows(be, 32), _rows(be, EE), _full((EE, 32)),
                      _full((1, 32)), _full((1, 32)), _full((32, 32)),
                      _full((1, 32))],
            out_specs=_rows(be, 32),
            out_shape=jax.ShapeDtypeStruct((EH, 32), F32),
        )(g_half, ea_half, w1l["w1e"], w1l["w1r"], w1l["b1"], w1l["w2"],
          w1l["b2"])

    g0a = _sc_gather_kernel(NCH, 0)(a0, b0, rowr, colr)
    g0b = _sc_gather_kernel(NCH, CB)(a0, b0, rowr, colr)
    m0a, eaa = edge0_half(g0a, 0)
    s0a = _sc_scatter_kernel(NCH, 0)(m0a, colr)
    m0b, eab = edge0_half(g0b, EH)
    s0b = _sc_scatter_kernel(NCH, CB)(m0b, colr)

    h2, p32n, a1, b1t = pl.pallas_call(
        _node0_body,
        grid=(gn,),
        in_specs=[_rows(bn, HD), _rows(bn, 32), _rows(bn, 32), _rows(bn, 32),
                  _rows(bn, 32), _rows(bn, 32),
                  _full((HD, 32)), _full((32, 32)), _full((1, 32)),
                  _full((32, HD)), _full((1, HD)), _full((1, HD)),
                  _full((1, HD)), _full((HD, 32)), _full((HD, 32))],
        out_specs=[_rows(bn, HD), _rows(bn, 32), _rows(bn, 32),
                   _rows(bn, 32)],
        out_shape=[jax.ShapeDtypeStruct((N, HD), F32),
                   jax.ShapeDtypeStruct((N, 32), F32),
                   jax.ShapeDtypeStruct((N, 32), F32),
                   jax.ShapeDtypeStruct((N, 32), F32)],
    )(h1, s0a[0][:N], s0a[1][:N], s0b[0][:N], s0b[1][:N], p32,
      w0["n1h"], w0["n1m"], w0["b1n"], w0["n2"],
      w0["b2n"], w0["gln"], w0["bln"], w1l["w1a"], w1l["w1b"])

    g1a = _sc_gather_kernel(NCH, 0)(a1, b1t, rowr, colr)
    g1b = _sc_gather_kernel(NCH, CB)(a1, b1t, rowr, colr)
    m1a = edge1_half(g1a, eaa, 0)
    s1a = _sc_scatter_kernel(NCH, 0)(m1a, colr)
    m1b = edge1_half(g1b, eab, EH)
    s1b = _sc_scatter_kernel(NCH, CB)(m1b, colr)

    h_out = pl.pallas_call(
        _node1_body,
        grid=(gn,),
        in_specs=[_rows(bn, HD), _rows(bn, 32), _rows(bn, 32), _rows(bn, 32),
                  _rows(bn, 32),
                  _full((HD, 32)), _full((32, 32)), _full((1, 32)),
                  _full((32, HD)), _full((1, HD)), _full((1, HD)),
                  _full((1, HD))],
        out_specs=_rows(bn, HD),
        out_shape=jax.ShapeDtypeStruct((N, HD), F32),
    )(h2, s1a[0][:N], s1a[1][:N], s1b[0][:N], s1b[1][:N],
      w1l["n1h"], w1l["n1m"], w1l["b1n"], w1l["n2"],
      w1l["b2n"], w1l["gln"], w1l["bln"])

    pos_out = p32n[:, 28:31]
    return (h_out, pos_out)


# speedup vs baseline: 1.0244x; 1.0244x over previous
import functools
import math

import jax
import jax.numpy as jnp
from jax import lax
from jax.experimental import pallas as pl
from jax.experimental.pallas import tpu as pltpu
from jax.experimental.pallas import tpu_sc as plsc

N = 10000
E = 160000
HD = 128
MSG = 24
EE = 16
THETA = 10000.0

NC = 2
NS = 16
NW = NC * NS
CH = 128
NCHUNK = 40
EP = NW * NCHUNK * CH
EPW = EP // NW
NP = 10240
F32 = jnp.float32


def _silu(x):
    return x / (1.0 + jnp.exp(-x))


def _dot(a, b):
    return jnp.dot(a.astype(jnp.bfloat16), b.astype(jnp.bfloat16),
                   preferred_element_type=F32)


def _bf(x):
    return x.astype(jnp.bfloat16).astype(F32)


_PIO2_HI = 1.5707963705062866
_PIO2_LO = -4.371138828673793e-08


def _fast_cos(x):
    q = jnp.floor(x * (2.0 / math.pi) + 0.5)
    r = (x - q * _PIO2_HI) - q * _PIO2_LO
    r2 = r * r
    c = 1.0 + r2 * (-0.5 + r2 * (4.16666679e-2 + r2 * (-1.38888881e-3
        + r2 * 2.48015873e-5)))
    s = r * (1.0 + r2 * (-1.66666672e-1 + r2 * (8.33333358e-3
        + r2 * (-1.98412701e-4 + r2 * 2.75573188e-6))))
    qi = q.astype(jnp.int32)
    val = jnp.where((qi & 1) == 1, s, c)
    return jnp.where(((qi + 1) & 2) == 0, val, -val)


def _prep_body(h_ref, p32_ref, wemb_ref, bemb_ref, w1a_ref, w1b_ref,
               h1_ref, a_ref, b_ref):
    h1 = _dot(h_ref[...], wemb_ref[...]) + bemb_ref[...]
    h1_ref[...] = h1
    p32 = p32_ref[...]
    a_ref[...] = p32 - _dot(h1, w1a_ref[...])
    b_ref[...] = p32 + _dot(h1, w1b_ref[...])


def _edge0_body(base, g_ref, w1e_ref, w1r_ref, b1_ref, w2_ref, b2_ref,
                c1_ref, c1b_ref, c2_ref, out_ref, ea_ref):
    g = g_ref[...]
    lane32 = lax.broadcasted_iota(jnp.int32, (1, 32), 1)
    posm = (lane32 >= 28) & (lane32 < 31)
    gp = jnp.where(posm, g, 0.0)
    radial = jnp.sum(gp * gp, axis=1, keepdims=True)
    dist = jnp.sqrt(radial)
    lane16 = lax.broadcasted_iota(jnp.int32, (1, 16), 1)
    f16 = jnp.exp(-math.log(THETA) * ((lane16 // 2) * 2).astype(F32) / EE)
    ang = dist * f16
    off = jnp.where(lane16 % 2 == 0, jnp.float32(math.pi / 2), 0.0)
    ea = _fast_cos(ang - off)
    ea_ref[...] = ea.astype(jnp.bfloat16)
    pre = (g + _dot(ea, w1e_ref[...]) + _bf(radial) * _bf(w1r_ref[...])
           + b1_ref[...])
    msg = _silu(_dot(_silu(pre), w2_ref[...]) + b2_ref[...])
    t = _silu(_dot(msg, c1_ref[...]) + c1b_ref[...])
    w = jnp.sum(_bf(t) * _bf(c2_ref[...]), axis=1, keepdims=True)
    out = (jnp.where(lane32 < 24, msg, 0.0)
           + jnp.where(lane32 == 24, 1.0, 0.0)
           + jnp.where(posm, w * g, 0.0))
    gid = (lax.broadcasted_iota(jnp.int32, (g.shape[0], 1), 0)
           + pl.program_id(0) * g.shape[0] + base)
    out_ref[...] = jnp.where(gid < E, out, 0.0)


def _edge1_body(base, g_ref, ea_ref, w1e_ref, w1r_ref, b1_ref, w2_ref,
                b2_ref, out_ref):
    g = g_ref[...]
    lane32 = lax.broadcasted_iota(jnp.int32, (1, 32), 1)
    posm = (lane32 >= 28) & (lane32 < 31)
    gp = jnp.where(posm, g, 0.0)
    radial = jnp.sum(gp * gp, axis=1, keepdims=True)
    pre = (g + _dot(ea_ref[...], w1e_ref[...])
           + _bf(radial) * _bf(w1r_ref[...]) + b1_ref[...])
    msg = _silu(_dot(_silu(pre), w2_ref[...]) + b2_ref[...])
    out = (jnp.where(lane32 < 24, msg, 0.0)
           + jnp.where(lane32 == 24, 1.0, 0.0))
    gid = (lax.broadcasted_iota(jnp.int32, (g.shape[0], 1), 0)
           + pl.program_id(0) * g.shape[0] + base)
    out_ref[...] = jnp.where(gid < E, out, 0.0)


def _node_core(hh, s, n1h, n1m, b1n, n2, b2n, gln, bln):
    lane32 = lax.broadcasted_iota(jnp.int32, (1, 32), 1)
    cnt = jnp.sum(jnp.where(lane32 == 24, s, 0.0), axis=1, keepdims=True)
    denom = jnp.maximum(cnt, 1.0)
    magg = jnp.where(lane32 < 24, s, 0.0) / denom
    u = _dot(hh, n1h) + _dot(magg, n1m) + b1n
    hn = hh + _dot(_silu(u), n2) + b2n
    m = jnp.mean(hn, axis=1, keepdims=True)
    v = jnp.mean((hn - m) ** 2, axis=1, keepdims=True)
    return (hn - m) / jnp.sqrt(v + 1e-5) * gln + bln


def _node0_body(h_ref, sa_ref, sb_ref, p32_ref, n1h_ref,
                n1m_ref, b1n_ref, n2_ref, b2n_ref, gln_ref, bln_ref,
                w1a_ref, w1b_ref, hout_ref, p32n_ref, a_ref, b_ref):
    s = (sa_ref[0] + sa_ref[1]) + (sb_ref[0] + sb_ref[1])
    ho = _node_core(h_ref[...], s, n1h_ref[...], n1m_ref[...], b1n_ref[...],
                    n2_ref[...], b2n_ref[...], gln_ref[...], bln_ref[...])
    hout_ref[...] = ho
    lane32 = lax.broadcasted_iota(jnp.int32, (1, 32), 1)
    cadd = jnp.where((lane32 >= 28) & (lane32 < 31), s, 0.0)
    p32n = p32_ref[...] + cadd
    p32n_ref[...] = p32n
    a_ref[...] = p32n - _dot(ho, w1a_ref[...])
    b_ref[...] = p32n + _dot(ho, w1b_ref[...])


def _node1_body(h_ref, sa_ref, sb_ref, n1h_ref, n1m_ref,
                b1n_ref, n2_ref, b2n_ref, gln_ref, bln_ref, hout_ref):
    s = (sa_ref[0] + sa_ref[1]) + (sb_ref[0] + sb_ref[1])
    hout_ref[...] = _node_core(h_ref[...], s, n1h_ref[...], n1m_ref[...],
                               b1n_ref[...], n2_ref[...], b2n_ref[...],
                               gln_ref[...], bln_ref[...])


def _full(shape):
    return pl.BlockSpec(shape, lambda i: (0,) * len(shape))


def _rows(bs, ncols):
    return pl.BlockSpec((bs, ncols), lambda i: (i, 0))


@functools.cache
def _sc_mesh():
    return plsc.VectorSubcoreMesh(core_axis_name="c", subcore_axis_name="s",
                                  num_cores=NC, num_subcores=NS)


@functools.cache
def _sc_gather_kernel(nchunk, cbase):
    epw = nchunk * CH

    def body(atab, btab, rowr, colr, g_out, idxr, idxc, bufa, bufb,
             sema, semb):
        cid = lax.axis_index("c")
        sid = lax.axis_index("s")
        wid = sid * NC + cid
        pltpu.sync_copy(rowr.at[pl.ds(cbase + wid * nchunk, nchunk)], idxr)
        pltpu.sync_copy(colr.at[pl.ds(cbase + wid * nchunk, nchunk)], idxc)

        def fire(j, b):
            pltpu.async_copy(atab.at[idxc.at[j]], bufa[b], sema[b])
            pltpu.async_copy(btab.at[idxr.at[j]], bufb[b], semb[b])

        def process(j, b):
            pltpu.make_async_copy(atab.at[idxc.at[j]], bufa[b],
                                  sema[b]).wait()
            pltpu.make_async_copy(btab.at[idxr.at[j]], bufb[b],
                                  semb[b]).wait()

            def sub(i, c2):
                bufb[b][i, 0:16] = bufb[b][i, 0:16] - bufa[b][i, 0:16]
                bufb[b][i, 16:32] = bufb[b][i, 16:32] - bufa[b][i, 16:32]
                return c2

            lax.fori_loop(0, CH, sub, 0, unroll=8)
            pltpu.sync_copy(bufb[b], g_out.at[pl.ds(wid * epw + j * CH, CH)])

        for b in range(4):
            fire(b, b)

        def group(t, carry):
            for b in range(4):
                j = t * 4 + b
                process(j, b)

                @pl.when(j + 4 < nchunk)
                def _():
                    fire(j + 4, b)

            return carry

        lax.fori_loop(0, nchunk // 4, group, 0)

    return pl.kernel(
        body,
        out_type=jax.ShapeDtypeStruct((NW * epw, 32), F32),
        mesh=_sc_mesh(),
        compiler_params=pltpu.CompilerParams(use_tc_tiling_on_sc=False),
        scratch_types=[
            pltpu.VMEM((nchunk, CH), jnp.int32),
            pltpu.VMEM((nchunk, CH), jnp.int32),
            [pltpu.VMEM((CH, 32), F32)] * 4,
            [pltpu.VMEM((CH, 32), F32)] * 4,
            [pltpu.SemaphoreType.DMA] * 4,
            [pltpu.SemaphoreType.DMA] * 4,
        ],
    )


@functools.cache
def _sc_scatter_kernel(nchunk, cbase):
    epw = nchunk * CH

    def body(m_hbm, colr, s_out, idxc, mbuf, zbuf, acc, msem):
        cid = lax.axis_index("c")
        sid = lax.axis_index("s")
        wid = sid * NC + cid
        rpt = NP // NS

        def zb(i, carry):
            zbuf[i, 0:16] = jnp.zeros((16,), F32)
            zbuf[i, 16:32] = jnp.zeros((16,), F32)
            return carry

        lax.fori_loop(0, rpt, zb, 0, unroll=8)
        pltpu.sync_copy(zbuf, acc.at[pl.ds(sid * rpt, rpt)])
        pltpu.sync_copy(colr.at[pl.ds(cbase + wid * nchunk, nchunk)], idxc)
        plsc.subcore_barrier()

        def fire(j, b):
            pltpu.async_copy(m_hbm.at[pl.ds(wid * epw + j * CH, CH)],
                             mbuf[b], msem[b])

        def process(j, b):
            pltpu.make_async_copy(
                m_hbm.at[pl.ds(wid * epw + j * CH, CH)], mbuf[b],
                msem[b]).wait()
            pltpu.sync_copy(mbuf[b], acc.at[idxc.at[j]], add=True)

        fire(0, 0)
        fire(1, 1)

        def pair(t, carry):
            j0 = t * 2
            process(j0, 0)

            @pl.when(j0 + 2 < nchunk)
            def _():
                fire(j0 + 2, 0)

            process(j0 + 1, 1)

            @pl.when(j0 + 3 < nchunk)
            def _():
                fire(j0 + 3, 1)

            return carry

        lax.fori_loop(0, nchunk // 2, pair, 0)
        plsc.subcore_barrier()
        pltpu.sync_copy(acc.at[pl.ds(sid * rpt, rpt)], zbuf)
        pltpu.sync_copy(zbuf, s_out.at[cid, pl.ds(sid * rpt, rpt)])

    return pl.kernel(
        body,
        out_type=jax.ShapeDtypeStruct((NC, NP, 32), F32),
        mesh=_sc_mesh(),
        compiler_params=pltpu.CompilerParams(use_tc_tiling_on_sc=False),
        scratch_types=[
            pltpu.VMEM((nchunk, CH), jnp.int32),
            [pltpu.VMEM((CH, 32), F32)] * 2,
            pltpu.VMEM((NP // NS, 32), F32),
            pltpu.VMEM_SHARED((NP, 32), F32),
            [pltpu.SemaphoreType.DMA] * 2,
        ],
    )


def _padc(w, cols=32):
    return jnp.zeros((w.shape[0], cols), F32).at[:, : w.shape[1]].set(w)


def _padrc(w, rows=32, cols=32):
    return (jnp.zeros((rows, cols), F32)
            .at[: w.shape[0], : w.shape[1]].set(w))


def _padv(b, cols=32):
    return jnp.zeros((1, cols), F32).at[0, : b.shape[0]].set(b)


def kernel(h, pos, edge_index, params):
    row = edge_index[0]
    col = edge_index[1]
    rowr = jnp.zeros((EP,), jnp.int32).at[:E].set(row).reshape(
        NW * NCHUNK, CH)
    colr = jnp.zeros((EP,), jnp.int32).at[:E].set(col).reshape(
        NW * NCHUNK, CH)
    p32 = jnp.zeros((N, 32), F32).at[:, 28:31].set(pos)

    lp = params["layers"]
    emb_w = params["node_emb"]["W"]
    emb_b = params["node_emb"]["b"].reshape(1, HD)

    def layer_weights(l):
        lay = lp[l]
        w1 = lay["msg1"]["W"]
        d = {
            "w1a": _padc(w1[:HD]),
            "w1b": _padc(w1[HD:2 * HD]),
            "w1e": _padc(w1[2 * HD:2 * HD + EE]),
            "w1r": _padv(w1[2 * HD + EE]),
            "b1": _padv(lay["msg1"]["b"]),
            "w2": _padrc(lay["msg2"]["W"]),
            "b2": _padv(lay["msg2"]["b"]),
            "n1h": _padc(lay["node1"]["W"][:HD]),
            "n1m": _padrc(lay["node1"]["W"][HD:]),
            "b1n": _padv(lay["node1"]["b"]),
            "n2": jnp.zeros((32, HD), F32).at[:MSG].set(lay["node2"]["W"]),
            "b2n": lay["node2"]["b"].reshape(1, HD),
            "gln": lay["ln_g"].reshape(1, HD),
            "bln": lay["ln_b"].reshape(1, HD),
        }
        if "coord1" in lay:
            d["c1"] = _padrc(lay["coord1"]["W"])
            d["c1b"] = _padv(lay["coord1"]["b"])
            d["c2"] = _padv(lay["coord2"]["W"][:, 0])
        return d

    w0 = layer_weights(0)
    w1l = layer_weights(1)

    bn = 1000
    gn = N // bn
    be = 2048
    EH = EP // 2
    geh = EH // be
    NCH = NCHUNK // 2
    CB = NW * NCH

    h1, a0, b0 = pl.pallas_call(
        _prep_body,
        grid=(gn,),
        in_specs=[_rows(bn, HD), _rows(bn, 32), _full((HD, HD)),
                  _full((1, HD)), _full((HD, 32)), _full((HD, 32))],
        out_specs=[_rows(bn, HD), _rows(bn, 32), _rows(bn, 32)],
        out_shape=[jax.ShapeDtypeStruct((N, HD), F32),
                   jax.ShapeDtypeStruct((N, 32), F32),
                   jax.ShapeDtypeStruct((N, 32), F32)],
    )(h, p32, emb_w, emb_b, w0["w1a"], w0["w1b"])

    def edge0_half(g_half, base):
        return pl.pallas_call(
            functools.partial(_edge0_body, base),
            grid=(geh,),
            in_specs=[_rows(be, 32), _full((EE, 32)), _full((1, 32)),
                      _full((1, 32)), _full((32, 32)), _full((1, 32)),
                      _full((32, 32)), _full((1, 32)), _full((1, 32))],
            out_specs=[_rows(be, 32), _rows(be, EE)],
            out_shape=[jax.ShapeDtypeStruct((EH, 32), F32),
                       jax.ShapeDtypeStruct((EH, EE), jnp.bfloat16)],
        )(g_half, w0["w1e"], w0["w1r"], w0["b1"], w0["w2"], w0["b2"],
          w0["c1"], w0["c1b"], w0["c2"])

    def edge1_half(g_half, ea_half, base):
        return pl.pallas_call(
            functools.partial(_edge1_body, base),
            grid=(geh,),
            in_specs=[_rows(be, 32), _rows(be, EE), _full((EE, 32)),
                      _full((1, 32)), _full((1, 32)), _full((32, 32)),
                      _full((1, 32))],
            out_specs=_rows(be, 32),
            out_shape=jax.ShapeDtypeStruct((EH, 32), F32),
        )(g_half, ea_half, w1l["w1e"], w1l["w1r"], w1l["b1"], w1l["w2"],
          w1l["b2"])

    g0a = _sc_gather_kernel(NCH, 0)(a0, b0, rowr, colr)
    g0b = _sc_gather_kernel(NCH, CB)(a0, b0, rowr, colr)
    m0a, eaa = edge0_half(g0a, 0)
    s0a = _sc_scatter_kernel(NCH, 0)(m0a, colr)
    m0b, eab = edge0_half(g0b, EH)
    s0b = _sc_scatter_kernel(NCH, CB)(m0b, colr)

    h2, p32n, a1, b1t = pl.pallas_call(
        _node0_body,
        grid=(gn,),
        in_specs=[_rows(bn, HD),
                  pl.BlockSpec((NC, bn, 32), lambda i: (0, i, 0)),
                  pl.BlockSpec((NC, bn, 32), lambda i: (0, i, 0)),
                  _rows(bn, 32),
                  _full((HD, 32)), _full((32, 32)), _full((1, 32)),
                  _full((32, HD)), _full((1, HD)), _full((1, HD)),
                  _full((1, HD)), _full((HD, 32)), _full((HD, 32))],
        out_specs=[_rows(bn, HD), _rows(bn, 32), _rows(bn, 32),
                   _rows(bn, 32)],
        out_shape=[jax.ShapeDtypeStruct((N, HD), F32),
                   jax.ShapeDtypeStruct((N, 32), F32),
                   jax.ShapeDtypeStruct((N, 32), F32),
                   jax.ShapeDtypeStruct((N, 32), F32)],
    )(h1, s0a, s0b, p32,
      w0["n1h"], w0["n1m"], w0["b1n"], w0["n2"],
      w0["b2n"], w0["gln"], w0["bln"], w1l["w1a"], w1l["w1b"])

    g1a = _sc_gather_kernel(NCH, 0)(a1, b1t, rowr, colr)
    g1b = _sc_gather_kernel(NCH, CB)(a1, b1t, rowr, colr)
    m1a = edge1_half(g1a, eaa, 0)
    s1a = _sc_scatter_kernel(NCH, 0)(m1a, colr)
    m1b = edge1_half(g1b, eab, EH)
    s1b = _sc_scatter_kernel(NCH, CB)(m1b, colr)

    h_out = pl.pallas_call(
        _node1_body,
        grid=(gn,),
        in_specs=[_rows(bn, HD),
                  pl.BlockSpec((NC, bn, 32), lambda i: (0, i, 0)),
                  pl.BlockSpec((NC, bn, 32), lambda i: (0, i, 0)),
                  _full((HD, 32)), _full((32, 32)), _full((1, 32)),
                  _full((32, HD)), _full((1, HD)), _full((1, HD)),
                  _full((1, HD))],
        out_specs=_rows(bn, HD),
        out_shape=jax.ShapeDtypeStruct((N, HD), F32),
    )(h2, s1a, s1b,
      w1l["n1h"], w1l["n1m"], w1l["b1n"], w1l["n2"],
      w1l["b2n"], w1l["gln"], w1l["bln"])

    pos_out = p32n[:, 28:31]
    return (h_out, pos_out)


# speedup vs baseline: 1.0532x; 1.0281x over previous
import functools
import math

import jax
import jax.numpy as jnp
from jax import lax
from jax.experimental import pallas as pl
from jax.experimental.pallas import tpu as pltpu
from jax.experimental.pallas import tpu_sc as plsc

N = 10000
E = 160000
HD = 128
MSG = 24
EE = 16
THETA = 10000.0

NC = 2
NS = 16
NW = NC * NS
CH = 128
NCHUNK = 40
EP = NW * NCHUNK * CH
EPW = EP // NW
NP = 10240
F32 = jnp.float32


def _silu(x):
    return x / (1.0 + jnp.exp(-x))


def _dot(a, b):
    return jnp.dot(a.astype(jnp.bfloat16), b.astype(jnp.bfloat16),
                   preferred_element_type=F32)


def _bf(x):
    return x.astype(jnp.bfloat16).astype(F32)


_PIO2_HI = 1.5707963705062866
_PIO2_LO = -4.371138828673793e-08


def _fast_cos(x):
    q = jnp.floor(x * (2.0 / math.pi) + 0.5)
    r = (x - q * _PIO2_HI) - q * _PIO2_LO
    r2 = r * r
    c = 1.0 + r2 * (-0.5 + r2 * (4.16666679e-2 + r2 * (-1.38888881e-3
        + r2 * 2.48015873e-5)))
    s = r * (1.0 + r2 * (-1.66666672e-1 + r2 * (8.33333358e-3
        + r2 * (-1.98412701e-4 + r2 * 2.75573188e-6))))
    qi = q.astype(jnp.int32)
    val = jnp.where((qi & 1) == 1, s, c)
    return jnp.where(((qi + 1) & 2) == 0, val, -val)


def _prep_body(h_ref, p32_ref, wemb_ref, bemb_ref, w1a_ref, w1b_ref,
               h1_ref, a_ref, b_ref):
    h1 = _dot(h_ref[...], wemb_ref[...]) + bemb_ref[...]
    h1_ref[...] = h1
    p32 = p32_ref[...]
    a_ref[...] = p32 - _dot(h1, w1a_ref[...])
    b_ref[...] = p32 + _dot(h1, w1b_ref[...])


def _edge0_body(base, g_ref, w1e_ref, w1r_ref, b1_ref, w2_ref, b2_ref,
                c1_ref, c1b_ref, c2_ref, out_ref, ea_ref):
    g = g_ref[...]
    lane32 = lax.broadcasted_iota(jnp.int32, (1, 32), 1)
    posm = (lane32 >= 28) & (lane32 < 31)
    gp = jnp.where(posm, g, 0.0)
    radial = jnp.sum(gp * gp, axis=1, keepdims=True)
    dist = jnp.sqrt(radial)
    lane16 = lax.broadcasted_iota(jnp.int32, (1, 16), 1)
    f16 = jnp.exp(-math.log(THETA) * ((lane16 // 2) * 2).astype(F32) / EE)
    ang = dist * f16
    off = jnp.where(lane16 % 2 == 0, jnp.float32(math.pi / 2), 0.0)
    ea = _fast_cos(ang - off)
    ea_ref[...] = ea.astype(jnp.bfloat16)
    pre = (g + _dot(ea, w1e_ref[...]) + _bf(radial) * _bf(w1r_ref[...])
           + b1_ref[...])
    msg = _silu(_dot(_silu(pre), w2_ref[...]) + b2_ref[...])
    t = _silu(_dot(msg, c1_ref[...]) + c1b_ref[...])
    w = jnp.sum(_bf(t) * _bf(c2_ref[...]), axis=1, keepdims=True)
    out = (jnp.where(lane32 < 24, msg, 0.0)
           + jnp.where(lane32 == 24, 1.0, 0.0)
           + jnp.where(posm, w * g, 0.0))
    gid = (lax.broadcasted_iota(jnp.int32, (g.shape[0], 1), 0)
           + pl.program_id(0) * g.shape[0] + base)
    out_ref[...] = jnp.where(gid < E, out, 0.0)


def _edge1_body(base, g_ref, ea_ref, w1e_ref, w1r_ref, b1_ref, w2_ref,
                b2_ref, out_ref):
    g = g_ref[...]
    lane32 = lax.broadcasted_iota(jnp.int32, (1, 32), 1)
    posm = (lane32 >= 28) & (lane32 < 31)
    gp = jnp.where(posm, g, 0.0)
    radial = jnp.sum(gp * gp, axis=1, keepdims=True)
    pre = (g + _dot(ea_ref[...], w1e_ref[...])
           + _bf(radial) * _bf(w1r_ref[...]) + b1_ref[...])
    msg = _silu(_dot(_silu(pre), w2_ref[...]) + b2_ref[...])
    out = (jnp.where(lane32 < 24, msg, 0.0)
           + jnp.where(lane32 == 24, 1.0, 0.0))
    gid = (lax.broadcasted_iota(jnp.int32, (g.shape[0], 1), 0)
           + pl.program_id(0) * g.shape[0] + base)
    out_ref[...] = jnp.where(gid < E, out, 0.0)


def _node_core(hh, s, n1h, n1m, b1n, n2, b2n, gln, bln):
    lane32 = lax.broadcasted_iota(jnp.int32, (1, 32), 1)
    cnt = jnp.sum(jnp.where(lane32 == 24, s, 0.0), axis=1, keepdims=True)
    denom = jnp.maximum(cnt, 1.0)
    magg = jnp.where(lane32 < 24, s, 0.0) / denom
    u = _dot(hh, n1h) + _dot(magg, n1m) + b1n
    hn = hh + _dot(_silu(u), n2) + b2n
    m = jnp.mean(hn, axis=1, keepdims=True)
    v = jnp.mean((hn - m) ** 2, axis=1, keepdims=True)
    return (hn - m) / jnp.sqrt(v + 1e-5) * gln + bln


def _node0_body(h_ref, sa_ref, sb_ref, p32_ref, n1h_ref,
                n1m_ref, b1n_ref, n2_ref, b2n_ref, gln_ref, bln_ref,
                w1a_ref, w1b_ref, hout_ref, p32n_ref, a_ref, b_ref):
    s = (sa_ref[0] + sa_ref[1]) + (sb_ref[0] + sb_ref[1])
    ho = _node_core(h_ref[...], s, n1h_ref[...], n1m_ref[...], b1n_ref[...],
                    n2_ref[...], b2n_ref[...], gln_ref[...], bln_ref[...])
    hout_ref[...] = ho
    lane32 = lax.broadcasted_iota(jnp.int32, (1, 32), 1)
    cadd = jnp.where((lane32 >= 28) & (lane32 < 31), s, 0.0)
    p32n = p32_ref[...] + cadd
    p32n_ref[...] = p32n
    a_ref[...] = p32n - _dot(ho, w1a_ref[...])
    b_ref[...] = p32n + _dot(ho, w1b_ref[...])


def _node1_body(h_ref, sa_ref, sb_ref, n1h_ref, n1m_ref,
                b1n_ref, n2_ref, b2n_ref, gln_ref, bln_ref, hout_ref):
    s = (sa_ref[0] + sa_ref[1]) + (sb_ref[0] + sb_ref[1])
    hout_ref[...] = _node_core(h_ref[...], s, n1h_ref[...], n1m_ref[...],
                               b1n_ref[...], n2_ref[...], b2n_ref[...],
                               gln_ref[...], bln_ref[...])


def _full(shape):
    return pl.BlockSpec(shape, lambda i: (0,) * len(shape))


def _rows(bs, ncols):
    return pl.BlockSpec((bs, ncols), lambda i: (i, 0))


@functools.cache
def _sc_mesh():
    return plsc.VectorSubcoreMesh(core_axis_name="c", subcore_axis_name="s",
                                  num_cores=NC, num_subcores=NS)


@functools.cache
def _sc_gather_kernel(nchunk, cbase):
    epw = nchunk * CH

    def body(atab, btab, rowr, colr, g_out, idxr, idxc, bufa, bufb,
             sema, semb):
        cid = lax.axis_index("c")
        sid = lax.axis_index("s")
        wid = sid * NC + cid
        pltpu.sync_copy(rowr.at[pl.ds(cbase + wid * nchunk, nchunk)], idxr)
        pltpu.sync_copy(colr.at[pl.ds(cbase + wid * nchunk, nchunk)], idxc)

        def fire(j, b):
            pltpu.async_copy(atab.at[idxc.at[j]], bufa[b], sema[b])
            pltpu.async_copy(btab.at[idxr.at[j]], bufb[b], semb[b])

        def process(j, b):
            pltpu.make_async_copy(atab.at[idxc.at[j]], bufa[b],
                                  sema[b]).wait()
            pltpu.make_async_copy(btab.at[idxr.at[j]], bufb[b],
                                  semb[b]).wait()

            def sub(i, c2):
                bufb[b][i, 0:16] = bufb[b][i, 0:16] - bufa[b][i, 0:16]
                bufb[b][i, 16:32] = bufb[b][i, 16:32] - bufa[b][i, 16:32]
                return c2

            lax.fori_loop(0, CH, sub, 0, unroll=8)
            pltpu.sync_copy(bufb[b], g_out.at[pl.ds(wid * epw + j * CH, CH)])

        for b in range(4):
            fire(b, b)

        def group(t, carry):
            for b in range(4):
                j = t * 4 + b
                process(j, b)

                @pl.when(j + 4 < nchunk)
                def _():
                    fire(j + 4, b)

            return carry

        lax.fori_loop(0, nchunk // 4, group, 0)

    return pl.kernel(
        body,
        out_type=jax.ShapeDtypeStruct((NW * epw, 32), F32),
        mesh=_sc_mesh(),
        compiler_params=pltpu.CompilerParams(use_tc_tiling_on_sc=False),
        scratch_types=[
            pltpu.VMEM((nchunk, CH), jnp.int32),
            pltpu.VMEM((nchunk, CH), jnp.int32),
            [pltpu.VMEM((CH, 32), F32)] * 4,
            [pltpu.VMEM((CH, 32), F32)] * 4,
            [pltpu.SemaphoreType.DMA] * 4,
            [pltpu.SemaphoreType.DMA] * 4,
        ],
    )


@functools.cache
def _sc_scatter_kernel(nchunk, cbase):
    epw = nchunk * CH

    def body(m_hbm, colr, s_out, idxc, mbuf, zbuf, acc, msem):
        cid = lax.axis_index("c")
        sid = lax.axis_index("s")
        wid = sid * NC + cid
        rpt = NP // NS

        def zb(i, carry):
            zbuf[i, 0:16] = jnp.zeros((16,), F32)
            zbuf[i, 16:32] = jnp.zeros((16,), F32)
            return carry

        lax.fori_loop(0, rpt, zb, 0, unroll=8)
        pltpu.sync_copy(zbuf, acc.at[pl.ds(sid * rpt, rpt)])
        pltpu.sync_copy(colr.at[pl.ds(cbase + wid * nchunk, nchunk)], idxc)
        plsc.subcore_barrier()

        def fire(j, b):
            pltpu.async_copy(m_hbm.at[pl.ds(wid * epw + j * CH, CH)],
                             mbuf[b], msem[b])

        def process(j, b):
            pltpu.make_async_copy(
                m_hbm.at[pl.ds(wid * epw + j * CH, CH)], mbuf[b],
                msem[b]).wait()
            pltpu.sync_copy(mbuf[b], acc.at[idxc.at[j]], add=True)

        fire(0, 0)
        fire(1, 1)

        def pair(t, carry):
            j0 = t * 2
            process(j0, 0)

            @pl.when(j0 + 2 < nchunk)
            def _():
                fire(j0 + 2, 0)

            process(j0 + 1, 1)

            @pl.when(j0 + 3 < nchunk)
            def _():
                fire(j0 + 3, 1)

            return carry

        lax.fori_loop(0, nchunk // 2, pair, 0)
        plsc.subcore_barrier()
        pltpu.sync_copy(acc.at[pl.ds(sid * rpt, rpt)], zbuf)
        pltpu.sync_copy(zbuf, s_out.at[cid, pl.ds(sid * rpt, rpt)])

    return pl.kernel(
        body,
        out_type=jax.ShapeDtypeStruct((NC, NP, 32), F32),
        mesh=_sc_mesh(),
        compiler_params=pltpu.CompilerParams(use_tc_tiling_on_sc=False),
        scratch_types=[
            pltpu.VMEM((nchunk, CH), jnp.int32),
            [pltpu.VMEM((CH, 32), F32)] * 2,
            pltpu.VMEM((NP // NS, 32), F32),
            pltpu.VMEM_SHARED((NP, 32), F32),
            [pltpu.SemaphoreType.DMA] * 2,
        ],
    )


def _padc(w, cols=32):
    return jnp.zeros((w.shape[0], cols), F32).at[:, : w.shape[1]].set(w)


def _padrc(w, rows=32, cols=32):
    return (jnp.zeros((rows, cols), F32)
            .at[: w.shape[0], : w.shape[1]].set(w))


def _padv(b, cols=32):
    return jnp.zeros((1, cols), F32).at[0, : b.shape[0]].set(b)


def kernel(h, pos, edge_index, params):
    row = edge_index[0]
    col = edge_index[1]
    rowr = jnp.zeros((EP,), jnp.int32).at[:E].set(row).reshape(
        NW * NCHUNK, CH)
    colr = jnp.zeros((EP,), jnp.int32).at[:E].set(col).reshape(
        NW * NCHUNK, CH)
    p32 = jnp.zeros((N, 32), F32).at[:, 28:31].set(pos)

    lp = params["layers"]
    emb_w = params["node_emb"]["W"]
    emb_b = params["node_emb"]["b"].reshape(1, HD)

    def layer_weights(l):
        lay = lp[l]
        w1 = lay["msg1"]["W"]
        d = {
            "w1a": _padc(w1[:HD]),
            "w1b": _padc(w1[HD:2 * HD]),
            "w1e": _padc(w1[2 * HD:2 * HD + EE]),
            "w1r": _padv(w1[2 * HD + EE]),
            "b1": _padv(lay["msg1"]["b"]),
            "w2": _padrc(lay["msg2"]["W"]),
            "b2": _padv(lay["msg2"]["b"]),
            "n1h": _padc(lay["node1"]["W"][:HD]),
            "n1m": _padrc(lay["node1"]["W"][HD:]),
            "b1n": _padv(lay["node1"]["b"]),
            "n2": jnp.zeros((32, HD), F32).at[:MSG].set(lay["node2"]["W"]),
            "b2n": lay["node2"]["b"].reshape(1, HD),
            "gln": lay["ln_g"].reshape(1, HD),
            "bln": lay["ln_b"].reshape(1, HD),
        }
        if "coord1" in lay:
            d["c1"] = _padrc(lay["coord1"]["W"])
            d["c1b"] = _padv(lay["coord1"]["b"])
            d["c2"] = _padv(lay["coord2"]["W"][:, 0])
        return d

    w0 = layer_weights(0)
    w1l = layer_weights(1)

    bn = 1000
    gn = N // bn
    be = 4096
    EH = EP // 2
    geh = EH // be
    NCH = NCHUNK // 2
    CB = NW * NCH

    h1, a0, b0 = pl.pallas_call(
        _prep_body,
        grid=(gn,),
        in_specs=[_rows(bn, HD), _rows(bn, 32), _full((HD, HD)),
                  _full((1, HD)), _full((HD, 32)), _full((HD, 32))],
        out_specs=[_rows(bn, HD), _rows(bn, 32), _rows(bn, 32)],
        out_shape=[jax.ShapeDtypeStruct((N, HD), F32),
                   jax.ShapeDtypeStruct((N, 32), F32),
                   jax.ShapeDtypeStruct((N, 32), F32)],
    )(h, p32, emb_w, emb_b, w0["w1a"], w0["w1b"])

    def edge0_half(g_half, base):
        return pl.pallas_call(
            functools.partial(_edge0_body, base),
            grid=(geh,),
            in_specs=[_rows(be, 32), _full((EE, 32)), _full((1, 32)),
                      _full((1, 32)), _full((32, 32)), _full((1, 32)),
                      _full((32, 32)), _full((1, 32)), _full((1, 32))],
            out_specs=[_rows(be, 32), _rows(be, EE)],
            out_shape=[jax.ShapeDtypeStruct((EH, 32), F32),
                       jax.ShapeDtypeStruct((EH, EE), jnp.bfloat16)],
        )(g_half, w0["w1e"], w0["w1r"], w0["b1"], w0["w2"], w0["b2"],
          w0["c1"], w0["c1b"], w0["c2"])

    def edge1_half(g_half, ea_half, base):
        return pl.pallas_call(
            functools.partial(_edge1_body, base),
            grid=(geh,),
            in_specs=[_rows(be, 32), _rows(be, EE), _full((EE, 32)),
                      _full((1, 32)), _full((1, 32)), _full((32, 32)),
                      _full((1, 32))],
            out_specs=_rows(be, 32),
            out_shape=jax.ShapeDtypeStruct((EH, 32), F32),
        )(g_half, ea_half, w1l["w1e"], w1l["w1r"], w1l["b1"], w1l["w2"],
          w1l["b2"])

    g0a = _sc_gather_kernel(NCH, 0)(a0, b0, rowr, colr)
    g0b = _sc_gather_kernel(NCH, CB)(a0, b0, rowr, colr)
    m0a, eaa = edge0_half(g0a, 0)
    s0a = _sc_scatter_kernel(NCH, 0)(m0a, colr)
    m0b, eab = edge0_half(g0b, EH)
    s0b = _sc_scatter_kernel(NCH, CB)(m0b, colr)

    h2, p32n, a1, b1t = pl.pallas_call(
        _node0_body,
        grid=(gn,),
        in_specs=[_rows(bn, HD),
                  pl.BlockSpec((NC, bn, 32), lambda i: (0, i, 0)),
                  pl.BlockSpec((NC, bn, 32), lambda i: (0, i, 0)),
                  _rows(bn, 32),
                  _full((HD, 32)), _full((32, 32)), _full((1, 32)),
                  _full((32, HD)), _full((1, HD)), _full((1, HD)),
                  _full((1, HD)), _full((HD, 32)), _full((HD, 32))],
        out_specs=[_rows(bn, HD), _rows(bn, 32), _rows(bn, 32),
                   _rows(bn, 32)],
        out_shape=[jax.ShapeDtypeStruct((N, HD), F32),
                   jax.ShapeDtypeStruct((N, 32), F32),
                   jax.ShapeDtypeStruct((N, 32), F32),
                   jax.ShapeDtypeStruct((N, 32), F32)],
    )(h1, s0a, s0b, p32,
      w0["n1h"], w0["n1m"], w0["b1n"], w0["n2"],
      w0["b2n"], w0["gln"], w0["bln"], w1l["w1a"], w1l["w1b"])

    g1a = _sc_gather_kernel(NCH, 0)(a1, b1t, rowr, colr)
    g1b = _sc_gather_kernel(NCH, CB)(a1, b1t, rowr, colr)
    m1a = edge1_half(g1a, eaa, 0)
    s1a = _sc_scatter_kernel(NCH, 0)(m1a, colr)
    m1b = edge1_half(g1b, eab, EH)
    s1b = _sc_scatter_kernel(NCH, CB)(m1b, colr)

    h_out = pl.pallas_call(
        _node1_body,
        grid=(gn,),
        in_specs=[_rows(bn, HD),
                  pl.BlockSpec((NC, bn, 32), lambda i: (0, i, 0)),
                  pl.BlockSpec((NC, bn, 32), lambda i: (0, i, 0)),
                  _full((HD, 32)), _full((32, 32)), _full((1, 32)),
                  _full((32, HD)), _full((1, HD)), _full((1, HD)),
                  _full((1, HD))],
        out_specs=_rows(bn, HD),
        out_shape=jax.ShapeDtypeStruct((N, HD), F32),
    )(h2, s1a, s1b,
      w1l["n1h"], w1l["n1m"], w1l["b1n"], w1l["n2"],
      w1l["b2n"], w1l["gln"], w1l["bln"])

    pos_out = p32n[:, 28:31]
    return (h_out, pos_out)
